# baseline (speedup 1.0000x reference)
import jax
import jax.numpy as jnp

N = 10000
E = 320000
G = 16
NUM_LAYERS = 3
TEMP = 1.0


def _layer_norm(h, s, b):
    mu = jnp.mean(h, axis=-1, keepdims=True)
    var = jnp.var(h, axis=-1, keepdims=True)
    return (h - mu) / jnp.sqrt(var + 1e-5) * s + b


def _gumbel_hard(logits, key, tau):
    uni = jax.random.uniform(key, logits.shape, minval=1e-6, maxval=1.0 - 1e-6)
    g = -jnp.log(-jnp.log(uni))
    y_soft = jax.nn.softmax((logits + g) / tau, axis=-1)
    y_hard = jax.nn.one_hot(jnp.argmax(y_soft, axis=-1), logits.shape[-1], dtype=y_soft.dtype)
    return y_hard + y_soft - jax.lax.stop_gradient(y_soft)


def kernel(x, edge_attr, y, params, edge_index, batch):
    u, v = edge_index[0], edge_index[1]
    env_ea = jax.nn.relu(edge_attr @ params["W_env_attr"] + params["b_env_attr"])
    act_ea = jax.nn.relu(edge_attr @ params["W_act_attr"] + params["b_act_attr"])
    h = jax.nn.relu(x @ params["W_node"] + params["b_node"])
    deg = jax.ops.segment_sum(jnp.ones((E,), jnp.float32), v, num_segments=N)
    deg = jnp.maximum(deg, 1.0)
    gkey = jax.random.key(1234)

    def action_net(p, hh):
        z = jax.nn.relu(hh @ p["W1"])
        msg = z[u] + act_ea
        agg = jax.ops.segment_sum(msg, v, num_segments=N) / deg[:, None]
        return agg @ p["W2"] + p["b2"]

    for i in range(NUM_LAYERS):
        hn = _layer_norm(h, params["ln_scale"], params["ln_bias"])
        in_logits = action_net(params["in_act"], hn)
        out_logits = action_net(params["out_act"], hn)
        in_probs = _gumbel_hard(in_logits, jax.random.fold_in(gkey, 2 * i), TEMP)
        out_probs = _gumbel_hard(out_logits, jax.random.fold_in(gkey, 2 * i + 1), TEMP)
        ew = in_probs[:, 0][v] * out_probs[:, 0][u]
        p = params["conv"][i]
        xn = hn @ p["W_nb"]
        msg = (xn[u] + env_ea) * ew[:, None]
        agg = jax.ops.segment_sum(msg, v, num_segments=N)
        out = jax.nn.relu(agg + hn @ p["W_root"] + p["b"])
        h = hn + out
    h = _layer_norm(h, params["ln_scale"], params["ln_bias"])
    h = h @ params["W_dec"] + params["b_dec"]
    gate = jax.nn.relu(h @ params["gate"]["W1"] + params["gate"]["b1"]) @ params["gate"]["W2"] + params["gate"]["b2"]
    gate = gate[:, 0]
    m = jax.ops.segment_max(gate, batch, num_segments=G)
    m = jnp.where(jnp.isfinite(m), m, 0.0)
    e = jnp.exp(gate - m[batch])
    denom = jax.ops.segment_sum(e, batch, num_segments=G)
    att = e / (denom[batch] + 1e-16)
    emb = jax.ops.segment_sum(h * att[:, None], batch, num_segments=G)
    z = emb
    Ws, bs = params["head"]["Ws"], params["head"]["bs"]
    for W, b in zip(Ws[:-1], bs[:-1]):
        z = jax.nn.relu(z @ W + b)
    pred = z @ Ws[-1] + bs[-1]
    loss = jnp.mean((pred - y) ** 2)
    return pred, loss


# speedup vs baseline: 1.0044x; 1.0044x over previous
import jax
import jax.numpy as jnp
import numpy as np
from jax.experimental import pallas as pl
from jax.experimental.pallas import tpu as pltpu

N = 10000
E = 320000
G = 16
ENV_DIM = 128
ACT_DIM = 64
D = 128
NUM_LAYERS = 3
TEMP = 1.0


def _layer_norm(h, s, b):
    mu = jnp.mean(h, axis=-1, keepdims=True)
    var = jnp.var(h, axis=-1, keepdims=True)
    return (h - mu) / jnp.sqrt(var + 1e-5) * s + b


def _gumbel_hard(logits, key, tau):
    uni = jax.random.uniform(key, logits.shape, minval=1e-6, maxval=1.0 - 1e-6)
    g = -jnp.log(-jnp.log(uni))
    y_soft = jax.nn.softmax((logits + g) / tau, axis=-1)
    y_hard = jax.nn.one_hot(jnp.argmax(y_soft, axis=-1), logits.shape[-1], dtype=y_soft.dtype)
    return y_hard + y_soft - jax.lax.stop_gradient(y_soft)


def _dot(a, b):
    return jax.lax.dot_general(a, b, (((1,), (0,)), ((), ())),
                               precision=jax.lax.Precision.HIGHEST,
                               preferred_element_type=jnp.float32)


def _head_kernel(h_ref, batch_ref, y_ref, wdec_ref, bdec_ref, gw1_ref, gb1_ref,
                 gw2_ref, gb2_ref, w0_ref, b0_ref, w1_ref, b1_ref, w2_ref, b2_ref,
                 w3_ref, b3_ref, w4_ref, b4_ref, w5_ref, b5_ref,
                 pred_ref, loss_ref):
    h = h_ref[...]
    hd = _dot(h, wdec_ref[...]) + bdec_ref[...]
    gate = _dot(jax.nn.relu(_dot(hd, gw1_ref[...]) + gb1_ref[...]), gw2_ref[...]) + gb2_ref[...]
    gate = gate[:, 0]
    batch = batch_ref[...]
    onehot = (batch == jax.lax.broadcasted_iota(jnp.int32, (1, G), 1)).astype(jnp.float32)
    neg = jnp.float32(-1e30)
    m = jnp.max(jnp.where(onehot > 0, gate[:, None], neg), axis=0)
    m = jnp.where(m > jnp.float32(-1e29), m, 0.0)
    e = jnp.exp(gate - _dot(onehot, m[:, None])[:, 0])
    denom = jnp.sum(e[:, None] * onehot, axis=0)
    att = e / (_dot(onehot, denom[:, None])[:, 0] + 1e-16)
    emb = jax.lax.dot_general(onehot, hd * att[:, None], (((0,), (0,)), ((), ())),
                              precision=jax.lax.Precision.HIGHEST,
                              preferred_element_type=jnp.float32)
    z = emb
    for w_ref, b_ref in ((w0_ref, b0_ref), (w1_ref, b1_ref), (w2_ref, b2_ref),
                         (w3_ref, b3_ref), (w4_ref, b4_ref)):
        z = jax.nn.relu(_dot(z, w_ref[...]) + b_ref[...])
    pred = _dot(z, w5_ref[...]) + b5_ref[...]
    pred_ref[...] = pred
    loss_ref[...] = jnp.mean((pred - y_ref[...]) ** 2).reshape(1, 1)


def kernel(x, edge_attr, y, params, edge_index, batch):
    u, v = edge_index[0], edge_index[1]
    env_ea = jax.nn.relu(edge_attr @ params["W_env_attr"] + params["b_env_attr"])
    act_ea = jax.nn.relu(edge_attr @ params["W_act_attr"] + params["b_act_attr"])
    h = jax.nn.relu(x @ params["W_node"] + params["b_node"])
    deg = jax.ops.segment_sum(jnp.ones((E,), jnp.float32), v, num_segments=N)
    deg = jnp.maximum(deg, 1.0)
    gkey = jax.random.key(1234)

    def action_net(p, hh):
        z = jax.nn.relu(hh @ p["W1"])
        msg = z[u] + act_ea
        agg = jax.ops.segment_sum(msg, v, num_segments=N) / deg[:, None]
        return agg @ p["W2"] + p["b2"]

    for i in range(NUM_LAYERS):
        hn = _layer_norm(h, params["ln_scale"], params["ln_bias"])
        in_logits = action_net(params["in_act"], hn)
        out_logits = action_net(params["out_act"], hn)
        in_probs = _gumbel_hard(in_logits, jax.random.fold_in(gkey, 2 * i), TEMP)
        out_probs = _gumbel_hard(out_logits, jax.random.fold_in(gkey, 2 * i + 1), TEMP)
        ew = in_probs[:, 0][v] * out_probs[:, 0][u]
        p = params["conv"][i]
        xn = hn @ p["W_nb"]
        msg = (xn[u] + env_ea) * ew[:, None]
        agg = jax.ops.segment_sum(msg, v, num_segments=N)
        out = jax.nn.relu(agg + hn @ p["W_root"] + p["b"])
        h = hn + out
    h = _layer_norm(h, params["ln_scale"], params["ln_bias"])

    hp = params["head"]
    pad = 10016 - N
    h_pad = jnp.pad(h, ((0, pad), (0, 0)))
    batch_pad = jnp.pad(batch, (0, pad), constant_values=G + 7)[:, None]
    head_dims = [D, 64, 32, 16, 8, 4, 1]

    args = [h_pad, batch_pad, y, params["W_dec"], params["b_dec"][None, :],
            params["gate"]["W1"], params["gate"]["b1"][None, :],
            params["gate"]["W2"], params["gate"]["b2"][None, :]]
    for i in range(6):
        args.append(hp["Ws"][i])
        args.append(hp["bs"][i][None, :])

    pred, loss = pl.pallas_call(
        _head_kernel,
        out_shape=(jax.ShapeDtypeStruct((G, 1), jnp.float32),
                   jax.ShapeDtypeStruct((1, 1), jnp.float32)),
    )(*args)
    return pred, loss[0, 0]


# speedup vs baseline: 3.4392x; 3.4240x over previous
import functools

import jax
import jax.numpy as jnp
from jax import lax
from jax.experimental import pallas as pl
from jax.experimental.pallas import tpu as pltpu
from jax.experimental.pallas import tpu_sc as plsc

N = 10000
E = 320000
G = 16
ENV_DIM = 128
ACT_DIM = 64
D = 128
NUM_LAYERS = 3
TEMP = 1.0

NW = 32
PER_W = E // NW
CH = 128
NFULL = PER_W // CH
TAIL = PER_W - NFULL * CH
NPAD = 10240
ROWS_PER_TILE = NPAD // 16
TRASH = 10200


def _layer_norm(h, s, b):
    mu = jnp.mean(h, axis=-1, keepdims=True)
    var = jnp.var(h, axis=-1, keepdims=True)
    return (h - mu) / jnp.sqrt(var + 1e-5) * s + b


def _dot(a, b):
    return jax.lax.dot_general(a, b, (((1,), (0,)), ((), ())),
                               precision=jax.lax.Precision.HIGHEST,
                               preferred_element_type=jnp.float32)


@functools.cache
def _build_seg(has_g: bool, has_env: bool):
    mesh = plsc.VectorSubcoreMesh(core_axis_name="c", subcore_axis_name="s")

    scratch = [
        pltpu.VMEM((CH,), jnp.int32),
        pltpu.VMEM((CH,), jnp.int32),
        pltpu.VMEM((TAIL,), jnp.int32),
        pltpu.VMEM((TAIL,), jnp.int32),
        pltpu.VMEM((16, 128), jnp.float32),
        pltpu.VMEM_SHARED((NPAD, 128), jnp.float32),
    ]
    if has_g:
        scratch.append(pltpu.VMEM((CH, 128), jnp.float32))
        scratch.append(pltpu.VMEM((TAIL, 128), jnp.float32))
        scratch.append(pltpu.SemaphoreType.DMA)
    if has_env:
        scratch.append(pltpu.VMEM((CH, 128), jnp.float32))
        scratch.append(pltpu.VMEM((TAIL, 128), jnp.float32))

    def body(*refs):
        i = 0
        g_hbm = env_hbm = None
        if has_g:
            g_hbm = refs[i]; i += 1
        if has_env:
            env_hbm = refs[i]; i += 1
        u_hbm = refs[i]; i += 1
        v_hbm = refs[i]; i += 1
        out_hbm = refs[i]; i += 1
        u_v = refs[i]; i += 1
        v_v = refs[i]; i += 1
        u_t = refs[i]; i += 1
        v_t = refs[i]; i += 1
        zb = refs[i]; i += 1
        acc = refs[i]; i += 1
        rows_v = rows_t = env_v = env_t = sem = None
        if has_g:
            rows_v = refs[i]; i += 1
            rows_t = refs[i]; i += 1
            sem = refs[i]; i += 1
        if has_env:
            env_v = refs[i]; i += 1
            env_t = refs[i]; i += 1

        c = lax.axis_index("c")
        s = lax.axis_index("s")
        w = c * 16 + s

        zvec = jnp.zeros((16,), jnp.float32)
        for r in range(16):
            for cc in range(8):
                zb[r, pl.ds(cc * 16, 16)] = zvec
        for j in range(ROWS_PER_TILE // 16):
            pltpu.sync_copy(zb, acc.at[pl.ds(s * ROWS_PER_TILE + j * 16, 16), :])
        plsc.subcore_barrier()

        base0 = w * PER_W

        def do_chunk(base, u_r, v_r, rows_r, env_r, sz):
            pltpu.sync_copy(u_hbm.at[pl.ds(base, sz)], u_r)
            pltpu.sync_copy(v_hbm.at[pl.ds(base, sz)], v_r)
            if has_g:
                pltpu.async_copy(g_hbm.at[u_r], rows_r, sem).wait()
                pltpu.sync_copy(rows_r, acc.at[v_r], add=True)
            if has_env:
                pltpu.sync_copy(env_hbm.at[pl.ds(base, sz), :], env_r)
                pltpu.sync_copy(env_r, acc.at[v_r], add=True)

        def loop_body(it, carry):
            base = pl.multiple_of(base0 + it * CH, 8)
            do_chunk(base, u_v, v_v, rows_v, env_v, CH)
            return carry

        lax.fori_loop(0, NFULL, loop_body, 0)
        do_chunk(pl.multiple_of(base0 + NFULL * CH, 8), u_t, v_t, rows_t, env_t,
                 TAIL)

        plsc.subcore_barrier()
        pltpu.sync_copy(acc.at[pl.ds(s * ROWS_PER_TILE, ROWS_PER_TILE), :],
                        out_hbm.at[c, pl.ds(s * ROWS_PER_TILE, ROWS_PER_TILE), :])

    return pl.kernel(
        body,
        out_type=jax.ShapeDtypeStruct((2, NPAD, 128), jnp.float32),
        mesh=mesh,
        scratch_types=scratch,
    )


def _seg_agg(u, v, g=None, env=None):
    k = _build_seg(g is not None, env is not None)
    args = []
    if g is not None:
        args.append(g)
    if env is not None:
        args.append(env)
    args.extend([u, v])
    parts = k(*args)
    return parts[0] + parts[1]


def _head_kernel(h_ref, batch_ref, y_ref, wdec_ref, bdec_ref, gw1_ref, gb1_ref,
                 gw2_ref, gb2_ref, w0_ref, b0_ref, w1_ref, b1_ref, w2_ref, b2_ref,
                 w3_ref, b3_ref, w4_ref, b4_ref, w5_ref, b5_ref,
                 pred_ref, loss_ref):
    h = h_ref[...]
    hd = _dot(h, wdec_ref[...]) + bdec_ref[...]
    gate = _dot(jax.nn.relu(_dot(hd, gw1_ref[...]) + gb1_ref[...]), gw2_ref[...]) + gb2_ref[...]
    gate = gate[:, 0]
    batch = batch_ref[...]
    onehot = (batch == jax.lax.broadcasted_iota(jnp.int32, (1, G), 1)).astype(jnp.float32)
    neg = jnp.float32(-1e30)
    m = jnp.max(jnp.where(onehot > 0, gate[:, None], neg), axis=0)
    m = jnp.where(m > jnp.float32(-1e29), m, 0.0)
    e = jnp.exp(gate - _dot(onehot, m[:, None])[:, 0])
    denom = jnp.sum(e[:, None] * onehot, axis=0)
    att = e / (_dot(onehot, denom[:, None])[:, 0] + 1e-16)
    emb = jax.lax.dot_general(onehot, hd * att[:, None], (((0,), (0,)), ((), ())),
                              precision=jax.lax.Precision.HIGHEST,
                              preferred_element_type=jnp.float32)
    z = emb
    for w_ref, b_ref in ((w0_ref, b0_ref), (w1_ref, b1_ref), (w2_ref, b2_ref),
                         (w3_ref, b3_ref), (w4_ref, b4_ref)):
        z = jax.nn.relu(_dot(z, w_ref[...]) + b_ref[...])
    pred = _dot(z, w5_ref[...]) + b5_ref[...]
    pred_ref[...] = pred
    loss_ref[...] = jnp.mean((pred - y_ref[...]) ** 2).reshape(1, 1)


def _head(h, batch, y, params):
    hp = params["head"]
    pad = 10016 - N
    h_pad = jnp.pad(h, ((0, pad), (0, 0)))
    batch_pad = jnp.pad(batch, (0, pad), constant_values=G + 7)[:, None]

    args = [h_pad, batch_pad, y, params["W_dec"], params["b_dec"][None, :],
            params["gate"]["W1"], params["gate"]["b1"][None, :],
            params["gate"]["W2"], params["gate"]["b2"][None, :]]
    for i in range(6):
        args.append(hp["Ws"][i])
        args.append(hp["bs"][i][None, :])

    pred, loss = pl.pallas_call(
        _head_kernel,
        out_shape=(jax.ShapeDtypeStruct((G, 1), jnp.float32),
                   jax.ShapeDtypeStruct((1, 1), jnp.float32)),
    )(*args)
    return pred, loss[0, 0]


def kernel(x, edge_attr, y, params, edge_index, batch):
    u, v = edge_index[0], edge_index[1]
    env_ea = jax.nn.relu(edge_attr @ params["W_env_attr"] + params["b_env_attr"])
    act_ea = jax.nn.relu(edge_attr @ params["W_act_attr"] + params["b_act_attr"])
    h = jax.nn.relu(x @ params["W_node"] + params["b_node"])

    pre_rows = jnp.concatenate(
        [act_ea, jnp.ones((E, 1), jnp.float32), jnp.zeros((E, 63), jnp.float32)],
        axis=1)
    pre = _seg_agg(u, v, env=pre_rows)
    s_act = pre[:N, :ACT_DIM]
    deg = jnp.maximum(pre[:N, ACT_DIM], 1.0)

    gkey = jax.random.key(1234)

    def gate_probs(logits, key):
        uni = jax.random.uniform(key, logits.shape, minval=1e-6, maxval=1.0 - 1e-6)
        g = -jnp.log(-jnp.log(uni))
        a = (logits + g) / TEMP
        return (a[:, 0] >= a[:, 1]).astype(jnp.float32)

    for i in range(NUM_LAYERS):
        hn = _layer_norm(h, params["ln_scale"], params["ln_bias"])
        z_in = jax.nn.relu(hn @ params["in_act"]["W1"])
        z_out = jax.nn.relu(hn @ params["out_act"]["W1"])
        aggz = _seg_agg(u, v, g=jnp.concatenate([z_in, z_out], axis=1))
        agg_in = (aggz[:N, :ACT_DIM] + s_act) / deg[:, None]
        agg_out = (aggz[:N, ACT_DIM:] + s_act) / deg[:, None]
        in_logits = agg_in @ params["in_act"]["W2"] + params["in_act"]["b2"]
        out_logits = agg_out @ params["out_act"]["W2"] + params["out_act"]["b2"]
        in0 = gate_probs(in_logits, jax.random.fold_in(gkey, 2 * i))
        out0 = gate_probs(out_logits, jax.random.fold_in(gkey, 2 * i + 1))
        p = params["conv"][i]
        xn = hn @ p["W_nb"]
        vg = jnp.where(out0[u] > 0.5, v, TRASH)
        accb = _seg_agg(u, vg, g=xn, env=env_ea)
        agg = in0[:, None] * accb[:N]
        out = jax.nn.relu(agg + hn @ p["W_root"] + p["b"])
        h = hn + out

    h = _layer_norm(h, params["ln_scale"], params["ln_bias"])
    return _head(h, batch, y, params)
